# Initial kernel scaffold; baseline (speedup 1.0000x reference)
import jax
import jax.numpy as jnp
from jax.experimental import pallas as pl


def kernel(x, edge_index, batch, exinfo, W1_root, W1_rel, b1, W2_root, W2_rel, b2, Wl1, bl1, Wl2, bl2, Wl3, bl3):
    raise NotImplementedError("write your pallas kernel here")


# speedup vs baseline: 9.1934x; 9.1934x over previous
import functools

import jax
import jax.numpy as jnp
from jax import lax
from jax.experimental import pallas as pl
from jax.experimental.pallas import tpu as pltpu
from jax.experimental.pallas import tpu_sc as plsc

_N = 10000
_E = 640000
_G = 64

_CH = 128
_NC = 2
_NS = 16
_NW = _NC * _NS
_CHUNKS = 157
_EPAD = _NW * _CH * _CHUNKS
_NPAD = 10112
_RPT = _NPAD // _NS


def _sc_segment_sum(table, srcp, dstp, zeros, d):
    mesh = plsc.VectorSubcoreMesh(core_axis_name="c", subcore_axis_name="s")

    @functools.partial(
        pl.kernel,
        out_type=jax.ShapeDtypeStruct((_NC, _NPAD, d), jnp.float32),
        mesh=mesh,
        compiler_params=pltpu.CompilerParams(use_tc_tiling_on_sc=False),
        scratch_types=[
            pltpu.VMEM((_CH,), jnp.int32),
            pltpu.VMEM((_CH,), jnp.int32),
            pltpu.VMEM((_CH, d), jnp.float32),
            pltpu.VMEM_SHARED((_NPAD, d), jnp.float32),
            pltpu.SemaphoreType.DMA,
        ],
    )
    def k(table_h, src_h, dst_h, zeros_h, out_h, src_v, dst_v, rows_v, acc_s, sem):
        cid = lax.axis_index("c")
        sid = lax.axis_index("s")
        wid = sid * _NC + cid
        base = wid * (_CH * _CHUNKS)
        r0 = pl.multiple_of(sid * _RPT, 8)

        pltpu.sync_copy(zeros_h.at[pl.ds(r0, _RPT), :],
                        acc_s.at[pl.ds(r0, _RPT), :])
        plsc.subcore_barrier()

        def body(j, _):
            off = base + j * _CH
            pltpu.sync_copy(src_h.at[pl.ds(off, _CH)], src_v)
            pltpu.sync_copy(dst_h.at[pl.ds(off, _CH)], dst_v)
            pltpu.async_copy(table_h.at[src_v], rows_v, sem).wait()
            pltpu.sync_copy(rows_v, acc_s.at[dst_v], add=True)
            return 0
        lax.fori_loop(0, _CHUNKS, body, 0)
        plsc.subcore_barrier()

        pltpu.sync_copy(acc_s.at[pl.ds(r0, _RPT), :],
                        out_h.at[cid, pl.ds(r0, _RPT), :])

    return k(table, srcp, dstp, zeros)


def _tc_layer1(x0p, parts, w_root, w_rel, b):
    nb = 1000
    grid = _N // nb

    def body(x_r, p_r, wr_r, wl_r, b_r, o_r):
        agg = p_r[0] + p_r[1]
        acc = jnp.dot(x_r[...], wr_r[...], preferred_element_type=jnp.float32)
        acc += jnp.dot(agg, wl_r[...], preferred_element_type=jnp.float32)
        o_r[...] = jnp.maximum(acc + b_r[...], 0.0)

    return pl.pallas_call(
        body,
        grid=(grid,),
        in_specs=[
            pl.BlockSpec((nb, 16), lambda i: (i, 0)),
            pl.BlockSpec((2, nb, 16), lambda i: (0, i, 0)),
            pl.BlockSpec((16, 128), lambda i: (0, 0)),
            pl.BlockSpec((16, 128), lambda i: (0, 0)),
            pl.BlockSpec((1, 128), lambda i: (0, 0)),
        ],
        out_specs=pl.BlockSpec((nb, 128), lambda i: (i, 0)),
        out_shape=jax.ShapeDtypeStruct((_N, 128), jnp.float32),
    )(x0p, parts, w_root, w_rel, b)


def _tc_layer2_head(x1, parts, batch3, expad, w_root, w_rel, b2,
                    wl1a, wl1b, bl1, wl2, bl2, wl3, bl3):
    nb = 1000
    grid = _N // nb

    def body(x1_r, p_r, bt_r, ex_r, wr_r, wl_r, b2_r,
             w1a_r, w1b_r, b1_r, w2_r, b2h_r, w3_r, b3_r, o_r, pool):
        i = pl.program_id(0)

        @pl.when(i == 0)
        def _init():
            pool[...] = jnp.full((_G, 256), -jnp.inf, jnp.float32)

        x1b = x1_r[...]
        agg = p_r[0] + p_r[1]
        x2 = jnp.dot(x1b, wr_r[...], preferred_element_type=jnp.float32)
        x2 += jnp.dot(agg, wl_r[...], preferred_element_type=jnp.float32)
        x2 = jnp.maximum(x2 + b2_r[...], 0.0)

        bt = bt_r[0]
        g0 = jnp.min(bt)
        g1 = jnp.max(bt)

        def upd(g, _):
            m = bt == g
            m1 = jnp.max(jnp.where(m, x1b, -jnp.inf), axis=0, keepdims=True)
            m2 = jnp.max(jnp.where(m, x2, -jnp.inf), axis=0, keepdims=True)
            row = jnp.concatenate([m1, m2], axis=1)
            pool[pl.ds(g, 1), :] = jnp.maximum(pool[pl.ds(g, 1), :], row)
            return 0
        lax.fori_loop(g0, g1 + 1, upd, 0)

        @pl.when(i == grid - 1)
        def _head():
            p = pool[...]
            p = jnp.where(jnp.isfinite(p), p, 0.0)
            h = jnp.dot(p, w1a_r[...], preferred_element_type=jnp.float32)
            h += jnp.dot(ex_r[...], w1b_r[...], preferred_element_type=jnp.float32)
            h = jnp.maximum(h + b1_r[...], 0.0)
            h = jnp.maximum(jnp.dot(h, w2_r[...], preferred_element_type=jnp.float32) + b2h_r[...], 0.0)
            z = jnp.dot(h, w3_r[...], preferred_element_type=jnp.float32) + b3_r[...]
            zm = z - jnp.max(z, axis=-1, keepdims=True)
            o_r[...] = zm - jnp.log(jnp.sum(jnp.exp(zm), axis=-1, keepdims=True))

    return pl.pallas_call(
        body,
        grid=(grid,),
        in_specs=[
            pl.BlockSpec((nb, 128), lambda i: (i, 0)),
            pl.BlockSpec((2, nb, 128), lambda i: (0, i, 0)),
            pl.BlockSpec((1, nb, 1), lambda i: (i, 0, 0)),
            pl.BlockSpec((_G, 16), lambda i: (0, 0)),
            pl.BlockSpec((128, 128), lambda i: (0, 0)),
            pl.BlockSpec((128, 128), lambda i: (0, 0)),
            pl.BlockSpec((1, 128), lambda i: (0, 0)),
            pl.BlockSpec((256, 64), lambda i: (0, 0)),
            pl.BlockSpec((16, 64), lambda i: (0, 0)),
            pl.BlockSpec((1, 64), lambda i: (0, 0)),
            pl.BlockSpec((64, 32), lambda i: (0, 0)),
            pl.BlockSpec((1, 32), lambda i: (0, 0)),
            pl.BlockSpec((32, 10), lambda i: (0, 0)),
            pl.BlockSpec((1, 10), lambda i: (0, 0)),
        ],
        out_specs=pl.BlockSpec((_G, 10), lambda i: (0, 0)),
        out_shape=jax.ShapeDtypeStruct((_G, 10), jnp.float32),
        scratch_shapes=[pltpu.VMEM((_G, 256), jnp.float32)],
    )(x1, parts, batch3, expad, w_root, w_rel, b2,
      wl1a, wl1b, bl1, wl2, bl2, wl3, bl3)


def kernel(x, edge_index, batch, exinfo, W1_root, W1_rel, b1,
           W2_root, W2_rel, b2, Wl1, bl1, Wl2, bl2, Wl3, bl3):
    src = edge_index[0]
    dst = edge_index[1]
    pad = _EPAD - _E
    srcp = jnp.concatenate([src, jnp.zeros((pad,), jnp.int32)])
    dstp = jnp.concatenate([dst, jnp.full((pad,), _N, jnp.int32)])

    x0p = jnp.pad(x[:, 2:5], ((0, 0), (0, 13)))
    w1r = jnp.pad(W1_root, ((0, 13), (0, 0)))
    w1l = jnp.pad(W1_rel, ((0, 13), (0, 0)))

    z16 = jnp.zeros((_NPAD, 16), jnp.float32)
    z128 = jnp.zeros((_NPAD, 128), jnp.float32)

    agg0 = _sc_segment_sum(x0p, srcp, dstp, z16, 16)
    x1 = _tc_layer1(x0p, agg0, w1r, w1l, b1.reshape(1, 128))

    agg1 = _sc_segment_sum(x1, srcp, dstp, z128, 128)

    batch3 = batch.reshape(_N // 1000, 1000, 1)
    expad = jnp.pad(exinfo, ((0, 0), (0, 6)))
    wl1a = Wl1[:256]
    wl1b = jnp.pad(Wl1[256:], ((0, 6), (0, 0)))

    return _tc_layer2_head(
        x1, agg1, batch3, expad, W2_root, W2_rel, b2.reshape(1, 128),
        wl1a, wl1b, bl1.reshape(1, 64), Wl2, bl2.reshape(1, 32),
        Wl3, bl3.reshape(1, 10))
